# baseline (speedup 1.0000x reference)
import functools

import jax
import jax.numpy as jnp
from jax import lax
from jax.experimental import pallas as pl
from jax.experimental.pallas import tpu as pltpu
from jax.experimental.pallas import tpu_sc as plsc

N = 10000
E = 320000
D_IN = 128
D_H = 128
D_OUT = 64
EPS = 1e-5

NC = 2
NS = 16
NW = NC * NS

SCHUNK = 128
S_NCH = E // SCHUNK
S_BASE = S_NCH // NW
S_TAIL_W = S_NCH % NW
S_ITERS = S_BASE // 6
S_NCH_PAD = S_NCH + 12

DCHUNK = 320
D_NCH = E // DCHUNK
D_STEPS = D_NCH // NW
D_EXTRA = D_NCH % NW

ROWS_PER_S = 624
TAIL0 = ROWS_PER_S * NS
TAIL = N - TAIL0

DEG_W = 128

_MESH = dict(core_axis_name="c", subcore_axis_name="s",
             num_cores=NC, num_subcores=NS)


def _zero_acc(zeros_hbm, acc, sid):
  pltpu.sync_copy(zeros_hbm, acc.at[pl.ds(sid * ROWS_PER_S, ROWS_PER_S)])
  @pl.when(sid == 0)
  def _():
    pltpu.sync_copy(zeros_hbm.at[pl.ds(0, TAIL)], acc.at[pl.ds(TAIL0, TAIL)])


def _write_out(acc, out_hbm, cid, sid):
  row0 = sid * ROWS_PER_S
  pltpu.sync_copy(acc.at[pl.ds(row0, ROWS_PER_S)],
                  out_hbm.at[cid, pl.ds(row0, ROWS_PER_S)])
  @pl.when(sid == 0)
  def _():
    pltpu.sync_copy(acc.at[pl.ds(TAIL0, TAIL)],
                    out_hbm.at[cid, pl.ds(TAIL0, TAIL)])


@functools.lru_cache(maxsize=None)
def _make_seg_sum(D):
  @functools.partial(
      pl.kernel,
      mesh=plsc.VectorSubcoreMesh(**_MESH),
      out_type=jax.ShapeDtypeStruct((NC, N, D), jnp.float32),
      scratch_types=[
          pltpu.VMEM((3, 2, SCHUNK), jnp.int32),
          pltpu.VMEM((3, 2, SCHUNK), jnp.int32),
          pltpu.VMEM((SCHUNK, D), jnp.float32),
          pltpu.VMEM((SCHUNK, D), jnp.float32),
          pltpu.VMEM((SCHUNK, D), jnp.float32),
          pltpu.VMEM_SHARED((N, D), jnp.float32),
          pltpu.SemaphoreType.DMA,
          pltpu.SemaphoreType.DMA,
          pltpu.SemaphoreType.DMA,
          pltpu.SemaphoreType.DMA,
          pltpu.SemaphoreType.DMA,
          pltpu.SemaphoreType.DMA,
          pltpu.SemaphoreType.DMA,
          pltpu.SemaphoreType.DMA,
          pltpu.SemaphoreType.DMA,
      ],
  )
  def seg_sum(table_hbm, eidx_hbm, zeros_hbm, out_hbm,
              ib0, ib1, rows_a, rows_b, rows_c, acc,
              ga, gb, gc, sa, sb, sc, ta, tb, tc):
    cid = lax.axis_index("c")
    sid = lax.axis_index("s")
    wid = cid * NS + sid
    cb = S_BASE * wid + jnp.minimum(wid, S_TAIL_W)

    _zero_acc(zeros_hbm, acc, sid)
    plsc.subcore_barrier()

    def wait_scat(rows, sem):
      pltpu.make_async_copy(rows, acc.at[ib0.at[0, 1]], sem).wait()

    def phase_a(c0, first):
      if not first:
        wait_scat(rows_a, ta)
      da = pltpu.async_copy(table_hbm.at[ib0.at[0, 0]], rows_a, ga)
      if not first:
        wait_scat(rows_b, tb)
      db = pltpu.async_copy(table_hbm.at[ib0.at[1, 0]], rows_b, gb)
      if not first:
        wait_scat(rows_c, tc)
      dc = pltpu.async_copy(table_hbm.at[ib0.at[2, 0]], rows_c, gc)
      pltpu.sync_copy(eidx_hbm.at[pl.ds(c0 + 3, 3)], ib1)
      da.wait()
      ea = pltpu.async_copy(rows_a, acc.at[ib0.at[0, 1]], sa, add=True)
      db.wait()
      eb = pltpu.async_copy(rows_b, acc.at[ib0.at[1, 1]], sb, add=True)
      dc.wait()
      ec = pltpu.async_copy(rows_c, acc.at[ib0.at[2, 1]], sc, add=True)
      return ea, eb, ec

    def phase_b(c0, ea, eb, ec):
      ea.wait()
      da = pltpu.async_copy(table_hbm.at[ib1.at[0, 0]], rows_a, ga)
      eb.wait()
      db = pltpu.async_copy(table_hbm.at[ib1.at[1, 0]], rows_b, gb)
      ec.wait()
      dc = pltpu.async_copy(table_hbm.at[ib1.at[2, 0]], rows_c, gc)
      pltpu.sync_copy(eidx_hbm.at[pl.ds(c0 + 6, 3)], ib0)
      da.wait()
      pltpu.async_copy(rows_a, acc.at[ib1.at[0, 1]], ta, add=True)
      db.wait()
      pltpu.async_copy(rows_b, acc.at[ib1.at[1, 1]], tb, add=True)
      dc.wait()
      pltpu.async_copy(rows_c, acc.at[ib1.at[2, 1]], tc, add=True)

    pltpu.sync_copy(eidx_hbm.at[pl.ds(cb, 3)], ib0)
    phase_b(cb, *phase_a(cb, True))

    def iteration(i, carry):
      c0 = cb + 6 * i
      phase_b(c0, *phase_a(c0, False))
      return carry

    lax.fori_loop(1, S_ITERS, iteration, 0)

    wait_scat(rows_a, ta)
    wait_scat(rows_b, tb)
    wait_scat(rows_c, tc)

    @pl.when(wid < S_TAIL_W)
    def _():
      pltpu.async_copy(table_hbm.at[ib0.at[0, 0]], rows_a, ga).wait()
      pltpu.sync_copy(rows_a, acc.at[ib0.at[0, 1]], add=True)

    plsc.subcore_barrier()
    _write_out(acc, out_hbm, cid, sid)

  return seg_sum


@functools.lru_cache(maxsize=None)
def _make_deg():
  @functools.partial(
      pl.kernel,
      mesh=plsc.VectorSubcoreMesh(**_MESH),
      out_type=jax.ShapeDtypeStruct((NC, N, DEG_W), jnp.float32),
      scratch_types=[
          pltpu.VMEM((DCHUNK,), jnp.int32),
          pltpu.VMEM((DCHUNK, DEG_W), jnp.float32),
          pltpu.VMEM_SHARED((N, DEG_W), jnp.float32),
      ],
  )
  def deg_kernel(dst_hbm, ones_hbm, zeros_hbm, out_hbm, idx_d, ones, acc):
    cid = lax.axis_index("c")
    sid = lax.axis_index("s")
    wid = cid * NS + sid

    _zero_acc(zeros_hbm, acc, sid)
    pltpu.sync_copy(ones_hbm, ones)
    plsc.subcore_barrier()

    def step(i, carry):
      base = (wid + i * NW) * DCHUNK
      pltpu.sync_copy(dst_hbm.at[pl.ds(base, DCHUNK)], idx_d)
      pltpu.sync_copy(ones, acc.at[idx_d], add=True)
      return carry

    n_steps = D_STEPS + jnp.where(wid < D_EXTRA, 1, 0)
    lax.fori_loop(0, n_steps, step, 0)
    plsc.subcore_barrier()
    _write_out(acc, out_hbm, cid, sid)

  return deg_kernel


def _bn_relu(h, g, b):
  m = jnp.mean(h, axis=0)
  d = h - m[None, :]
  v = jnp.mean(d * d, axis=0)
  return jnp.maximum(d * lax.rsqrt(v + EPS)[None, :] * g[None, :] + b[None, :],
                     0.0)


def _matT(a, w):
  return lax.dot_general(a, w, (((1,), (1,)), ((), ())),
                         preferred_element_type=jnp.float32)


def _dense1_body(s_ref, dg_ref, x_ref, wl_ref, bl_ref, wr_ref, g_ref, be_ref,
                 h_out, inv_out):
  deg = dg_ref[0, :, 0:16] + dg_ref[1, :, 0:16]
  inv = 1.0 / jnp.maximum(deg, 1.0)
  inv_out[...] = inv
  agg = (s_ref[0] + s_ref[1]) * inv[:, 0:1]
  h = _matT(agg, wl_ref[...]) + bl_ref[...][None, :] + _matT(x_ref[...], wr_ref[...])
  h_out[...] = _bn_relu(h, g_ref[...], be_ref[...])


def _dense2_body(s_ref, h1_ref, inv_ref, wl_ref, bl_ref, wr_ref, g_ref, be_ref,
                 w3r_ref, h2_out, r_out):
  agg = (s_ref[0] + s_ref[1]) * inv_ref[...][:, 0:1]
  h = _matT(agg, wl_ref[...]) + bl_ref[...][None, :] + _matT(h1_ref[...], wr_ref[...])
  h2 = _bn_relu(h, g_ref[...], be_ref[...])
  h2_out[...] = h2
  r_out[...] = _matT(h2, w3r_ref[...])


def _dense3_body(s_ref, r_ref, inv_ref, w3l_ref, bl_ref, p_out):
  agg = (s_ref[0] + s_ref[1]) * inv_ref[...][:, 0:1]
  logits = _matT(agg, w3l_ref[...]) + bl_ref[...][None, :] + r_ref[...]
  mx = jnp.max(logits, axis=-1, keepdims=True)
  e = jnp.exp(logits - mx)
  p_out[...] = e / jnp.sum(e, axis=-1, keepdims=True)


_dense1 = pl.pallas_call(
    _dense1_body,
    out_shape=[jax.ShapeDtypeStruct((N, D_H), jnp.float32),
               jax.ShapeDtypeStruct((N, 16), jnp.float32)],
)

_dense2 = pl.pallas_call(
    _dense2_body,
    out_shape=[jax.ShapeDtypeStruct((N, D_H), jnp.float32),
               jax.ShapeDtypeStruct((N, D_OUT), jnp.float32)],
)

_dense3 = pl.pallas_call(
    _dense3_body,
    out_shape=jax.ShapeDtypeStruct((N, D_OUT), jnp.float32),
)


def kernel(x, edge_index, W1l, b1l, W1r, g1, be1, W2l, b2l, W2r, g2, be2,
           W3l, b3l, W3r):
  dst = edge_index[1]
  eidx = jnp.pad(
      edge_index.reshape(2, S_NCH, SCHUNK).transpose(1, 0, 2),
      ((0, S_NCH_PAD - S_NCH), (0, 0), (0, 0)))

  z128 = jnp.zeros((ROWS_PER_S, D_H), jnp.float32)
  zdeg = jnp.zeros((ROWS_PER_S, DEG_W), jnp.float32)
  ones = jnp.ones((DCHUNK, DEG_W), jnp.float32)
  seg = _make_seg_sum(D_H)

  dg = _make_deg()(dst, ones, zdeg)
  s1 = seg(x, eidx, z128)
  h1, inv = _dense1(s1, dg, x, W1l, b1l, W1r, g1, be1)

  s2 = seg(h1, eidx, z128)
  h2, r = _dense2(s2, h1, inv, W2l, b2l, W2r, g2, be2, W3r)

  s3 = seg(h2, eidx, z128)
  return _dense3(s3, r, inv, W3l, b3l)


# speedup vs baseline: 1.0485x; 1.0485x over previous
import functools

import jax
import jax.numpy as jnp
from jax import lax
from jax.experimental import pallas as pl
from jax.experimental.pallas import tpu as pltpu
from jax.experimental.pallas import tpu_sc as plsc

N = 10000
E = 320000
D_IN = 128
D_H = 128
D_OUT = 64
EPS = 1e-5

NC = 2
NS = 16
NW = NC * NS

SCHUNK = 128
S_NCH = E // SCHUNK
S_BASE = S_NCH // NW
S_TAIL_W = S_NCH % NW
S_ITERS = S_BASE // 6
S_NCH_PAD = S_NCH + 12

ROWS_PER_S = 624
TAIL0 = ROWS_PER_S * NS
TAIL = N - TAIL0

DEG_W = 128

_MESH = dict(core_axis_name="c", subcore_axis_name="s",
             num_cores=NC, num_subcores=NS)


def _zero_acc(zeros_hbm, acc, sid):
  pltpu.sync_copy(zeros_hbm, acc.at[pl.ds(sid * ROWS_PER_S, ROWS_PER_S)])
  @pl.when(sid == 0)
  def _():
    pltpu.sync_copy(zeros_hbm.at[pl.ds(0, TAIL)], acc.at[pl.ds(TAIL0, TAIL)])


def _write_out(acc, out_hbm, cid, sid):
  row0 = sid * ROWS_PER_S
  pltpu.sync_copy(acc.at[pl.ds(row0, ROWS_PER_S)],
                  out_hbm.at[cid, pl.ds(row0, ROWS_PER_S)])
  @pl.when(sid == 0)
  def _():
    pltpu.sync_copy(acc.at[pl.ds(TAIL0, TAIL)],
                    out_hbm.at[cid, pl.ds(TAIL0, TAIL)])


@functools.lru_cache(maxsize=None)
def _make_seg_sum(D, with_deg=False):
  out_type = jax.ShapeDtypeStruct((NC, N, D), jnp.float32)
  if with_deg:
    out_type = [out_type, jax.ShapeDtypeStruct((NC, N, DEG_W), jnp.float32)]

  @functools.partial(
      pl.kernel,
      mesh=plsc.VectorSubcoreMesh(**_MESH),
      out_type=out_type,
      scratch_types=[
          pltpu.VMEM((3, 2, SCHUNK), jnp.int32),
          pltpu.VMEM((3, 2, SCHUNK), jnp.int32),
          pltpu.VMEM((SCHUNK, D), jnp.float32),
          pltpu.VMEM((SCHUNK, D), jnp.float32),
          pltpu.VMEM((SCHUNK, D), jnp.float32),
          pltpu.VMEM_SHARED((N, D), jnp.float32),
          pltpu.SemaphoreType.DMA,
          pltpu.SemaphoreType.DMA,
          pltpu.SemaphoreType.DMA,
          pltpu.SemaphoreType.DMA,
          pltpu.SemaphoreType.DMA,
          pltpu.SemaphoreType.DMA,
          pltpu.SemaphoreType.DMA,
          pltpu.SemaphoreType.DMA,
          pltpu.SemaphoreType.DMA,
      ],
  )
  def seg_sum(table_hbm, eidx_hbm, zeros_hbm, *refs):
    if with_deg:
      (ones_hbm, out_hbm, dg_hbm,
       ib0, ib1, rows_a, rows_b, rows_c, acc,
       ga, gb, gc, sa, sb, sc, ta, tb, tc) = refs
    else:
      (out_hbm,
       ib0, ib1, rows_a, rows_b, rows_c, acc,
       ga, gb, gc, sa, sb, sc, ta, tb, tc) = refs
    cid = lax.axis_index("c")
    sid = lax.axis_index("s")
    wid = cid * NS + sid
    cb = S_BASE * wid + jnp.minimum(wid, S_TAIL_W)

    _zero_acc(zeros_hbm, acc, sid)
    plsc.subcore_barrier()

    def wait_scat(rows, sem):
      pltpu.make_async_copy(rows, acc.at[ib0.at[0, 1]], sem).wait()

    def phase_a(c0, first):
      if not first:
        wait_scat(rows_a, ta)
      da = pltpu.async_copy(table_hbm.at[ib0.at[0, 0]], rows_a, ga)
      if not first:
        wait_scat(rows_b, tb)
      db = pltpu.async_copy(table_hbm.at[ib0.at[1, 0]], rows_b, gb)
      if not first:
        wait_scat(rows_c, tc)
      dc = pltpu.async_copy(table_hbm.at[ib0.at[2, 0]], rows_c, gc)
      pltpu.sync_copy(eidx_hbm.at[pl.ds(c0 + 3, 3)], ib1)
      da.wait()
      ea = pltpu.async_copy(rows_a, acc.at[ib0.at[0, 1]], sa, add=True)
      db.wait()
      eb = pltpu.async_copy(rows_b, acc.at[ib0.at[1, 1]], sb, add=True)
      dc.wait()
      ec = pltpu.async_copy(rows_c, acc.at[ib0.at[2, 1]], sc, add=True)
      return ea, eb, ec

    def phase_b(c0, ea, eb, ec):
      ea.wait()
      da = pltpu.async_copy(table_hbm.at[ib1.at[0, 0]], rows_a, ga)
      eb.wait()
      db = pltpu.async_copy(table_hbm.at[ib1.at[1, 0]], rows_b, gb)
      ec.wait()
      dc = pltpu.async_copy(table_hbm.at[ib1.at[2, 0]], rows_c, gc)
      pltpu.sync_copy(eidx_hbm.at[pl.ds(c0 + 6, 3)], ib0)
      da.wait()
      pltpu.async_copy(rows_a, acc.at[ib1.at[0, 1]], ta, add=True)
      db.wait()
      pltpu.async_copy(rows_b, acc.at[ib1.at[1, 1]], tb, add=True)
      dc.wait()
      pltpu.async_copy(rows_c, acc.at[ib1.at[2, 1]], tc, add=True)

    pltpu.sync_copy(eidx_hbm.at[pl.ds(cb, 3)], ib0)
    phase_b(cb, *phase_a(cb, True))

    def iteration(i, carry):
      c0 = cb + 6 * i
      phase_b(c0, *phase_a(c0, False))
      return carry

    lax.fori_loop(1, S_ITERS, iteration, 0)

    wait_scat(rows_a, ta)
    wait_scat(rows_b, tb)
    wait_scat(rows_c, tc)

    @pl.when(wid < S_TAIL_W)
    def _():
      pltpu.async_copy(table_hbm.at[ib0.at[0, 0]], rows_a, ga).wait()
      pltpu.sync_copy(rows_a, acc.at[ib0.at[0, 1]], add=True)

    plsc.subcore_barrier()
    _write_out(acc, out_hbm, cid, sid)

    if with_deg:
      _zero_acc(zeros_hbm, acc, sid)
      pltpu.sync_copy(ones_hbm, rows_a)
      plsc.subcore_barrier()

      def dscat(ib, j, sem):
        pltpu.async_copy(rows_a, acc.at[ib.at[j, 1]], sem, add=True)

      pltpu.sync_copy(eidx_hbm.at[pl.ds(cb, 3)], ib0)
      dscat(ib0, 0, sa)
      dscat(ib0, 1, sb)
      dscat(ib0, 2, sc)
      pltpu.sync_copy(eidx_hbm.at[pl.ds(cb + 3, 3)], ib1)

      def dphase(ib, cnext, ibnext):
        wait_scat(rows_a, sa)
        dscat(ib, 0, sa)
        wait_scat(rows_a, sb)
        dscat(ib, 1, sb)
        wait_scat(rows_a, sc)
        dscat(ib, 2, sc)
        pltpu.sync_copy(eidx_hbm.at[pl.ds(cnext, 3)], ibnext)

      def diter(i, carry):
        c0 = cb + 6 * i
        dphase(ib1, c0 + 6, ib0)
        dphase(ib0, c0 + 9, ib1)
        return carry

      lax.fori_loop(0, S_ITERS - 1, diter, 0)
      dphase(ib1, cb + S_BASE, ib0)
      wait_scat(rows_a, sa)
      wait_scat(rows_a, sb)
      wait_scat(rows_a, sc)
      @pl.when(wid < S_TAIL_W)
      def _():
        pltpu.sync_copy(rows_a, acc.at[ib0.at[0, 1]], add=True)

      plsc.subcore_barrier()
      _write_out(acc, dg_hbm, cid, sid)

  return seg_sum


def _bn_relu(h, g, b):
  m = jnp.mean(h, axis=0)
  d = h - m[None, :]
  v = jnp.mean(d * d, axis=0)
  return jnp.maximum(d * lax.rsqrt(v + EPS)[None, :] * g[None, :] + b[None, :],
                     0.0)


def _matT(a, w):
  return lax.dot_general(a, w, (((1,), (1,)), ((), ())),
                         preferred_element_type=jnp.float32)


def _dense1_body(s_ref, dg_ref, x_ref, wl_ref, bl_ref, wr_ref, g_ref, be_ref,
                 h_out, inv_out):
  deg = dg_ref[0, :, 0:16] + dg_ref[1, :, 0:16]
  inv = 1.0 / jnp.maximum(deg, 1.0)
  inv_out[...] = inv
  agg = (s_ref[0] + s_ref[1]) * inv[:, 0:1]
  h = _matT(agg, wl_ref[...]) + bl_ref[...][None, :] + _matT(x_ref[...], wr_ref[...])
  h_out[...] = _bn_relu(h, g_ref[...], be_ref[...])


def _dense2_body(s_ref, h1_ref, inv_ref, wl_ref, bl_ref, wr_ref, g_ref, be_ref,
                 w3r_ref, h2_out, r_out):
  agg = (s_ref[0] + s_ref[1]) * inv_ref[...][:, 0:1]
  h = _matT(agg, wl_ref[...]) + bl_ref[...][None, :] + _matT(h1_ref[...], wr_ref[...])
  h2 = _bn_relu(h, g_ref[...], be_ref[...])
  h2_out[...] = h2
  r_out[...] = _matT(h2, w3r_ref[...])


def _dense3_body(s_ref, r_ref, inv_ref, w3l_ref, bl_ref, p_out):
  agg = (s_ref[0] + s_ref[1]) * inv_ref[...][:, 0:1]
  logits = _matT(agg, w3l_ref[...]) + bl_ref[...][None, :] + r_ref[...]
  mx = jnp.max(logits, axis=-1, keepdims=True)
  e = jnp.exp(logits - mx)
  p_out[...] = e / jnp.sum(e, axis=-1, keepdims=True)


_dense1 = pl.pallas_call(
    _dense1_body,
    out_shape=[jax.ShapeDtypeStruct((N, D_H), jnp.float32),
               jax.ShapeDtypeStruct((N, 16), jnp.float32)],
)

_dense2 = pl.pallas_call(
    _dense2_body,
    out_shape=[jax.ShapeDtypeStruct((N, D_H), jnp.float32),
               jax.ShapeDtypeStruct((N, D_OUT), jnp.float32)],
)

_dense3 = pl.pallas_call(
    _dense3_body,
    out_shape=jax.ShapeDtypeStruct((N, D_OUT), jnp.float32),
)


def kernel(x, edge_index, W1l, b1l, W1r, g1, be1, W2l, b2l, W2r, g2, be2,
           W3l, b3l, W3r):
  eidx = jnp.pad(
      edge_index.reshape(2, S_NCH, SCHUNK).transpose(1, 0, 2),
      ((0, S_NCH_PAD - S_NCH), (0, 0), (0, 0)))

  z128 = jnp.zeros((ROWS_PER_S, D_H), jnp.float32)
  ones = jnp.ones((SCHUNK, DEG_W), jnp.float32)
  seg = _make_seg_sum(D_H)

  s1, dg = _make_seg_sum(D_H, True)(x, eidx, z128, ones)
  h1, inv = _dense1(s1, dg, x, W1l, b1l, W1r, g1, be1)

  s2 = seg(h1, eidx, z128)
  h2, r = _dense2(s2, h1, inv, W2l, b2l, W2r, g2, be2, W3r)

  s3 = seg(h2, eidx, z128)
  return _dense3(s3, r, inv, W3l, b3l)
